# baseline (speedup 1.0000x reference)
import numpy as np
import jax
import jax.numpy as jnp
from jax.experimental import pallas as pl
from jax.experimental.pallas import tpu as pltpu

_B = 8
_TH = _TW = 32
_KH = _KW = 32
_PH = _PW = 8
_FH = _FW = 256
_DO = 64
_NI = _KH // _PH
_QP = _TH + _NI - 1


def _inv_counts_perm() -> np.ndarray:
    h0 = np.arange(_TH) * _PH
    dh = np.arange(_KH)
    rows = np.clip(h0[:, None] + dh[None, :], 0, _FH - 1)
    cnt = np.bincount(rows.ravel(), minlength=_FH).astype(np.float64)
    inv2d = 1.0 / (cnt[:, None] * cnt[None, :])
    perm = inv2d.reshape(_TH, _PH, _TW, _PW).transpose(0, 2, 1, 3)
    return np.ascontiguousarray(perm.reshape(_TH, _TW, _PH * _PW)).astype(np.float32)


_INVP = _inv_counts_perm()


_BB = 8


def _decoder_kernel(x_ref, w_ref, inv_ref, o_ref, acc_ref):
    for b in range(_BB):
        x = x_ref[b]

        acc_ref[...] = jnp.zeros((_QP, _QP, _PH * _PW), jnp.float32)
        for i in range(_NI):
            for j in range(_NI):
                wk = w_ref[_PH * i:_PH * (i + 1),
                           _PW * j:_PW * (j + 1), :].reshape(_PH * _PW, _DO)
                pk = jax.lax.dot_general(
                    x, wk, (((1,), (1,)), ((), ())),
                    preferred_element_type=jnp.float32)
                acc_ref[i:i + _TH, j:j + _TW, :] += (
                    pk.reshape(_TH, _TW, _PH * _PW))

        over_r = acc_ref[_TH:, :, :].reshape(
            _NI - 1, _QP, _PH, _PW).sum(axis=(0, 2))
        onehot7 = (jnp.arange(_PH) == _PH - 1).astype(jnp.float32)
        row_add = onehot7[None, :, None] * over_r[:, None, :]
        acc_ref[_TH - 1, :, :] += row_add.reshape(_QP, _PH * _PW)

        over_c = acc_ref[:_TH, _TW:, :].reshape(
            _TH, _NI - 1, _PH, _PW).sum(axis=(1, 3))
        col_add = over_c[:, :, None] * onehot7[None, None, :]
        acc_ref[:_TH, _TW - 1, :] += col_add.reshape(_TH, _PH * _PW)

        scaled = acc_ref[:_TH, :_TW, :] * inv_ref[...]
        field = scaled.reshape(_TH, _TW, _PH, _PW).transpose(0, 2, 1, 3)
        o_ref[b] = field.reshape(_FH, _FW)


def kernel(tgt, W):
    invp = jnp.asarray(_INVP)

    out = pl.pallas_call(
        _decoder_kernel,
        grid=(_B // _BB,),
        in_specs=[
            pl.BlockSpec((_BB, _TH * _TW, _DO), lambda b: (b, 0, 0)),
            pl.BlockSpec((_KH, _KW, _DO), lambda b: (0, 0, 0)),
            pl.BlockSpec((_TH, _TW, _PH * _PW), lambda b: (0, 0, 0)),
        ],
        out_specs=pl.BlockSpec((_BB, _FH, _FW), lambda b: (b, 0, 0)),
        out_shape=jax.ShapeDtypeStruct((_B, _FH, _FW), jnp.float32),
        scratch_shapes=[pltpu.VMEM((_QP, _QP, _PH * _PW), jnp.float32)],
        compiler_params=pltpu.CompilerParams(
            dimension_semantics=("parallel",)),
    )(tgt, W, invp)
    return out


# speedup vs baseline: 1.0220x; 1.0220x over previous
import numpy as np
import jax
import jax.numpy as jnp
from jax.experimental import pallas as pl
from jax.experimental.pallas import tpu as pltpu

_B = 8
_TH = _TW = 32
_KH = _KW = 32
_PH = _PW = 8
_FH = _FW = 256
_DO = 64
_NI = _KH // _PH
_QP = _TH + _NI - 1


def _inv_counts_perm() -> np.ndarray:
    h0 = np.arange(_TH) * _PH
    dh = np.arange(_KH)
    rows = np.clip(h0[:, None] + dh[None, :], 0, _FH - 1)
    cnt = np.bincount(rows.ravel(), minlength=_FH).astype(np.float64)
    inv2d = 1.0 / (cnt[:, None] * cnt[None, :])
    perm = inv2d.reshape(_TH, _PH, _TW, _PW).transpose(0, 2, 1, 3)
    return np.ascontiguousarray(
        perm.reshape(_TH, _TW, _PH * _PW)).astype(np.float32)


_INVP = _inv_counts_perm()


_BB = 4


def _decoder_kernel(x_ref, w_ref, inv_ref, o_ref, acc0_ref, acc1_ref):
    for b in range(_BB):
        acc_ref = acc0_ref if b % 2 == 0 else acc1_ref
        x = x_ref[b]

        acc_ref[...] = jnp.zeros((_QP, _QP, _PH * _PW), jnp.float32)
        for i in range(_NI):
            for j in range(_NI):
                wk = w_ref[_PH * i:_PH * (i + 1),
                           _PW * j:_PW * (j + 1), :].reshape(_PH * _PW, _DO)
                pk = jax.lax.dot_general(
                    x, wk, (((1,), (1,)), ((), ())),
                    preferred_element_type=jnp.float32)
                acc_ref[i:i + _TH, j:j + _TW, :] += (
                    pk.reshape(_TH, _TW, _PH * _PW))

        over_r = acc_ref[_TH:, :, :].reshape(
            _NI - 1, _QP, _PH, _PW).sum(axis=(0, 2))
        onehot7 = (jnp.arange(_PH) == _PH - 1).astype(jnp.float32)
        row_add = onehot7[None, :, None] * over_r[:, None, :]
        acc_ref[_TH - 1, :, :] += row_add.reshape(_QP, _PH * _PW)

        over_c = acc_ref[:_TH, _TW:, :].reshape(
            _TH, _NI - 1, _PH, _PW).sum(axis=(1, 3))
        col_add = over_c[:, :, None] * onehot7[None, None, :]
        acc_ref[:_TH, _TW - 1, :] += col_add.reshape(_TH, _PH * _PW)

        scaled = acc_ref[:_TH, :_TW, :] * inv_ref[...]
        field = scaled.reshape(_TH, _TW, _PH, _PW).transpose(0, 2, 1, 3)
        o_ref[b] = field.reshape(_FH, _FW)


def kernel(tgt, W):
    invp = jnp.asarray(_INVP)

    out = pl.pallas_call(
        _decoder_kernel,
        grid=(_B // _BB,),
        in_specs=[
            pl.BlockSpec((_BB, _TH * _TW, _DO), lambda b: (b, 0, 0)),
            pl.BlockSpec((_KH, _KW, _DO), lambda b: (0, 0, 0)),
            pl.BlockSpec((_TH, _TW, _PH * _PW), lambda b: (0, 0, 0)),
        ],
        out_specs=pl.BlockSpec((_BB, _FH, _FW), lambda b: (b, 0, 0)),
        out_shape=jax.ShapeDtypeStruct((_B, _FH, _FW), jnp.float32),
        scratch_shapes=[pltpu.VMEM((_QP, _QP, _PH * _PW), jnp.float32),
                        pltpu.VMEM((_QP, _QP, _PH * _PW), jnp.float32)],
        compiler_params=pltpu.CompilerParams(
            dimension_semantics=("parallel",)),
    )(tgt, W, invp)
    return out


# speedup vs baseline: 1.0236x; 1.0015x over previous
import numpy as np
import jax
import jax.numpy as jnp
from jax.experimental import pallas as pl
from jax.experimental.pallas import tpu as pltpu

_B = 8
_TH = _TW = 32
_KH = _KW = 32
_PH = _PW = 8
_FH = _FW = 256
_DO = 64
_NI = _KH // _PH
_QP = _TH + _NI - 1


def _inv_counts_perm() -> np.ndarray:
    h0 = np.arange(_TH) * _PH
    dh = np.arange(_KH)
    rows = np.clip(h0[:, None] + dh[None, :], 0, _FH - 1)
    cnt = np.bincount(rows.ravel(), minlength=_FH).astype(np.float64)
    inv2d = 1.0 / (cnt[:, None] * cnt[None, :])
    perm = inv2d.reshape(_TH, _PH, _TW, _PW).transpose(0, 2, 1, 3)
    return np.ascontiguousarray(
        perm.reshape(_TH, _TW, _PH * _PW)).astype(np.float32)


_INVP = _inv_counts_perm()


_BB = 4


def _decoder_kernel(x_ref, w_ref, inv_ref, o_ref, acc0_ref, acc1_ref):
    for b in range(_BB):
        acc_ref = acc0_ref if b % 2 == 0 else acc1_ref
        x = x_ref[b]

        acc_ref[...] = jnp.zeros((_QP, _QP, _PH * _PW), jnp.float32)
        for i in range(_NI):
            for j in range(_NI):
                wk = w_ref[_PH * i:_PH * (i + 1),
                           _PW * j:_PW * (j + 1), :].reshape(_PH * _PW, _DO)
                pk = jax.lax.dot_general(
                    x, wk, (((1,), (1,)), ((), ())),
                    preferred_element_type=jnp.float32)
                acc_ref[i:i + _TH, j:j + _TW, :] += (
                    pk.reshape(_TH, _TW, _PH * _PW))

        over_r = acc_ref[_TH:, :, :].reshape(
            _NI - 1, _QP, _PH, _PW).sum(axis=(0, 2))
        onehot7 = (jnp.arange(_PH) == _PH - 1).astype(jnp.float32)
        row_add = onehot7[None, :, None] * over_r[:, None, :]
        acc_ref[_TH - 1, :, :] += row_add.reshape(_QP, _PH * _PW)

        over_c = acc_ref[:_TH, _TW:, :].reshape(
            _TH, _NI - 1, _PH, _PW).sum(axis=(1, 3))
        col_add = over_c[:, :, None] * onehot7[None, None, :]
        acc_ref[:_TH, _TW - 1, :] += col_add.reshape(_TH, _PH * _PW)

        scaled = acc_ref[:_TH, :_TW, :] * inv_ref[...]
        field = scaled.reshape(_TH, _TW, _PH, _PW).transpose(0, 2, 1, 3)
        o_ref[b] = field.reshape(_FH, _FW)


def kernel(tgt, W):
    invp = jnp.asarray(_INVP)

    out = pl.pallas_call(
        _decoder_kernel,
        grid=(_B // _BB,),
        in_specs=[
            pl.BlockSpec((_BB, _TH * _TW, _DO), lambda b: (b, 0, 0)),
            pl.BlockSpec((_KH, _KW, _DO), lambda b: (0, 0, 0)),
            pl.BlockSpec((_TH, _TW, _PH * _PW), lambda b: (0, 0, 0)),
        ],
        out_specs=pl.BlockSpec((_BB, _FH, _FW), lambda b: (b, 0, 0)),
        out_shape=jax.ShapeDtypeStruct((_B, _FH, _FW), jnp.float32),
        scratch_shapes=[pltpu.VMEM((_QP, _QP, _PH * _PW), jnp.float32),
                        pltpu.VMEM((_QP, _QP, _PH * _PW), jnp.float32)],
        compiler_params=pltpu.CompilerParams(
            dimension_semantics=("parallel",),
            skip_device_barrier=True),
    )(tgt, W, invp)
    return out
